# baseline (speedup 1.0000x reference)
import functools

import jax
import jax.numpy as jnp
from jax import lax
from jax.experimental import pallas as pl
from jax.experimental.pallas import tpu as pltpu
from jax.experimental.pallas import tpu_sc as plsc

N, E, D = 10000, 160000, 256
NC, NS, L = 2, 16, 16
NW = NC * NS
EPT = E // NW
NG = 313
RPT = 312
MAGIC = 3361
LAST_R = N - (NW - 1) * RPT
ACC_R = 330
TRASH_LOC = 328
NBK = NW + 1
CW = 48
BQ = 64
CAP_W = 7680
PKS = 9

_f32 = jnp.float32
_i32 = jnp.int32

_GDN = lax.GatherDimensionNumbers(offset_dims=(), collapsed_slice_dims=(0,),
                                  start_index_map=(0,))


def _vgather(vec, idx):
    return lax.gather(vec, idx[:, None], _GDN, (1,),
                      mode=lax.GatherScatterMode.PROMISE_IN_BOUNDS)


def _rquant(cnt):
    return ((cnt + BQ + 15) >> 6) << 6


def _sc_bin_body(src_hbm, dst_hbm, lx_hbm, cov_hbm,
                 sv, dv, covv, listv):
    c = lax.axis_index("c")
    s = lax.axis_index("s")
    w = c * NS + s
    ebase = w * EPT

    lane = lax.iota(_i32, L)
    zero16 = jnp.zeros((L,), _i32)

    pltpu.sync_copy(src_hbm.at[pl.ds(pl.multiple_of(ebase, 8), EPT)],
                    sv.at[pl.ds(0, EPT)])
    pltpu.sync_copy(dst_hbm.at[pl.ds(pl.multiple_of(ebase, 8), EPT)],
                    dv.at[pl.ds(0, EPT)])

    def bucket_of(g, d):
        b = jnp.minimum((d * MAGIC) >> 20, NW - 1)
        gflag = jnp.where(g == NG - 1, 1, 0)
        tail = jnp.where(lane >= 8, gflag, 0)
        return jnp.where(tail > 0, NW, b)

    def hist_add(b, h0, h1, h2):
        for e in range(L):
            bc = jnp.full((L,), b[e], _i32)
            h0 = h0 + jnp.where(lane == bc, 1, 0)
            h1 = h1 + jnp.where(lane + L == bc, 1, 0)
            h2 = h2 + jnp.where(lane + 2 * L == bc, 1, 0)
        return h0, h1, h2

    def pass_a(g, carry):
        h0, h1, h2 = carry
        d = dv[pl.ds(g * L, L)]
        b = bucket_of(g, d)
        return hist_add(b, h0, h1, h2)

    c0, c1, c2 = lax.fori_loop(0, NG, pass_a, (zero16, zero16, zero16))

    cvs = [c0, c1, c2]
    offs = []
    off_acc = jnp.int32(0)
    for b in range(NBK):
        offs.append(off_acc)
        off_acc = off_acc + _rquant(cvs[b // L][b % L])

    o0, o1, o2 = zero16, zero16, zero16
    for b in range(NBK):
        sel = jnp.where(lane == (b % L), offs[b], 0)
        if b // L == 0:
            o0 = o0 + sel
        elif b // L == 1:
            o1 = o1 + sel
        else:
            o2 = o2 + sel

    covv[pl.ds(0, L)] = (o0 << 16) | c0
    covv[pl.ds(L, L)] = (o1 << 16) | c1
    covv[pl.ds(2 * L, L)] = (o2 << 16) | c2
    pltpu.sync_copy(covv, cov_hbm.at[pl.ds(w * CW, CW)])

    def pass_b(g, carry):
        r0, r1, r2 = carry
        d = dv[pl.ds(g * L, L)]
        sc = sv[pl.ds(g * L, L)]
        b = bucket_of(g, d)
        bi = b & (L - 1)
        pk = (sc << PKS) | ((d - b * RPT) & ((1 << PKS) - 1))
        s0 = _vgather(r0, bi)
        s1 = _vgather(r1, bi)
        s2 = _vgather(r2, bi)
        slot = jnp.where(b < L, s0, jnp.where(b < 2 * L, s1, s2))
        t0 = _vgather(o0, bi)
        t1 = _vgather(o1, bi)
        t2 = _vgather(o2, bi)
        obase = jnp.where(b < L, t0, jnp.where(b < 2 * L, t1, t2))
        adj = zero16
        for e in range(L):
            bc = jnp.full((L,), b[e], _i32)
            adj = adj + jnp.where((b == bc) & (lane > e), 1, 0)
        pos = obase + slot + adj
        for e in range(L):
            listv[pl.ds(pos[e], L)] = jnp.full((L,), pk[e], _i32)
        r0, r1, r2 = hist_add(b, r0, r1, r2)
        return (r0, r1, r2)

    r0, r1, r2 = lax.fori_loop(0, NG, pass_b, (zero16, zero16, zero16))

    trash16 = jnp.full((L,), TRASH_LOC, _i32)
    rvs = [r0, r1, r2]
    for b in range(NBK):
        cnt_b = rvs[b // L][b % L]
        o_b = offs[b]
        oend = o_b + _rquant(cnt_b)
        pos0 = o_b + cnt_b
        for k in range(5):
            p2 = pos0 + k * L

            @pl.when(p2 <= oend - L)
            def _():
                listv[pl.ds(p2, L)] = trash16
        listv[pl.ds(oend - L, L)] = trash16

    pltpu.sync_copy(listv.at[pl.ds(0, CAP_W)],
                    lx_hbm.at[pl.ds(w * CAP_W, CAP_W)])


def _make_sc_bin():
    mesh = plsc.VectorSubcoreMesh(core_axis_name="c", subcore_axis_name="s")
    return pl.kernel(
        _sc_bin_body,
        out_type=(jax.ShapeDtypeStruct((NW * CAP_W,), _i32),
                  jax.ShapeDtypeStruct((NW * CW,), _i32)),
        mesh=mesh,
        scratch_types=(pltpu.VMEM((EPT + L,), _i32),
                       pltpu.VMEM((EPT + L,), _i32),
                       pltpu.VMEM((CW,), _i32),
                       pltpu.VMEM((CAP_W + L,), _i32)))


_sc_bin = _make_sc_bin()


def _sc_deg_body(lx_hbm, cov_hbm, deg_hbm, dacc, pv, locb, cov):
    c = lax.axis_index("c")
    s = lax.axis_index("s")
    w = c * NS + s
    zero16 = jnp.zeros((L,), _f32)
    one16 = jnp.ones((L,), _f32)

    def zd(i, _):
        dacc[i, :] = zero16
        return 0
    lax.fori_loop(0, ACC_R, zd, 0)

    pltpu.sync_copy(cov_hbm, cov.at[pl.ds(0, NW * CW)])

    for wsrc in range(NW):
        v = cov[pl.ds(wsrc * CW + w, L)][0]
        cnt = v & 0xFFFF
        off = v >> 16
        nb = jnp.where(cnt > 0, (cnt + BQ + 15) >> 6, 0)

        def batch(k, _):
            p0 = pl.multiple_of(wsrc * CAP_W + off + k * BQ, 8)
            pltpu.sync_copy(lx_hbm.at[pl.ds(p0, BQ)], pv)
            for g in range(BQ // L):
                vv = pv[pl.ds(g * L, L)]
                locb[pl.ds(g * L, L)] = vv & ((1 << PKS) - 1)

            def grp(g, _):
                lv = locb[pl.ds(g * L, L)]
                for l in range(L):
                    plsc.addupdate(dacc.at[lv[l]], one16)
                return 0
            lax.fori_loop(0, BQ // L, grp, 0)
            return 0
        lax.fori_loop(0, nb, batch, 0)

    obase = pl.multiple_of(w * RPT, 8)

    @pl.when(w < NW - 1)
    def _():
        pltpu.sync_copy(dacc.at[pl.ds(0, RPT)], deg_hbm.at[pl.ds(obase, RPT)])

    @pl.when(w == NW - 1)
    def _():
        pltpu.sync_copy(dacc.at[pl.ds(0, LAST_R)],
                        deg_hbm.at[pl.ds(obase, LAST_R)])


def _make_sc_deg():
    mesh = plsc.VectorSubcoreMesh(core_axis_name="c", subcore_axis_name="s")
    return pl.kernel(
        _sc_deg_body,
        out_type=(jax.ShapeDtypeStruct((N, L), _f32),),
        mesh=mesh,
        scratch_types=(pltpu.VMEM((ACC_R, L), _f32),
                       pltpu.VMEM((BQ,), _i32),
                       pltpu.VMEM((BQ,), _i32),
                       pltpu.VMEM((NW * CW + L,), _i32)))


_sc_deg = _make_sc_deg()


def _sc_agg_body(h_hbm, lx_hbm, cov_hbm, agg_hbm,
                 acc, rows, pv, srcb, locb, cov):
    c = lax.axis_index("c")
    s = lax.axis_index("s")
    w = c * NS + s

    zero16 = jnp.zeros((L,), _f32)

    def za(i, _):
        acc[i >> 4, pl.ds((i & 15) * L, L)] = zero16
        return 0
    lax.fori_loop(0, ACC_R * 16, za, 0)

    pltpu.sync_copy(cov_hbm, cov.at[pl.ds(0, NW * CW)])

    def per_src(wsrc, _):
        v = cov[pl.ds(wsrc * CW + w, L)][0]
        cnt = v & 0xFFFF
        off = v >> 16
        nb = jnp.where(cnt > 0, (cnt + BQ + 15) >> 6, 0)

        def batch(k, _):
            p0 = pl.multiple_of(wsrc * CAP_W + off + k * BQ, 8)
            pltpu.sync_copy(lx_hbm.at[pl.ds(p0, BQ)], pv)
            for g in range(BQ // L):
                vv = pv[pl.ds(g * L, L)]
                srcb[pl.ds(g * L, L)] = vv >> PKS
                locb[pl.ds(g * L, L)] = vv & ((1 << PKS) - 1)
            pltpu.sync_copy(h_hbm.at[srcb], rows)

            def grp(g, _):
                lv = locb[pl.ds(g * L, L)]
                for l in range(L):
                    r = lv[l]
                    for j in range(D // L):
                        plsc.addupdate(acc.at[r, pl.ds(j * L, L)],
                                       rows[g * L + l, pl.ds(j * L, L)])
                return 0
            lax.fori_loop(0, BQ // L, grp, 0)
            return 0
        lax.fori_loop(0, nb, batch, 0)
        return 0
    lax.fori_loop(0, NW, per_src, 0)

    obase = pl.multiple_of(w * RPT, 8)

    @pl.when(w < NW - 1)
    def _():
        pltpu.sync_copy(acc.at[pl.ds(0, RPT)], agg_hbm.at[pl.ds(obase, RPT)])

    @pl.when(w == NW - 1)
    def _():
        pltpu.sync_copy(acc.at[pl.ds(0, LAST_R)],
                        agg_hbm.at[pl.ds(obase, LAST_R)])


def _make_sc_agg():
    mesh = plsc.VectorSubcoreMesh(core_axis_name="c", subcore_axis_name="s")
    return pl.kernel(
        _sc_agg_body,
        out_type=(jax.ShapeDtypeStruct((N, D), _f32),),
        mesh=mesh,
        scratch_types=(pltpu.VMEM((ACC_R, D), _f32),
                       pltpu.VMEM((BQ, D), _f32),
                       pltpu.VMEM((BQ,), _i32),
                       pltpu.VMEM((BQ,), _i32),
                       pltpu.VMEM((BQ,), _i32),
                       pltpu.VMEM((NW * CW + L,), _i32)))


_sc_agg = _make_sc_agg()

BLK = 1000


def _lin_body(x_ref, w_ref, b_ref, o_ref):
    o_ref[...] = (jnp.dot(x_ref[...], w_ref[...],
                          preferred_element_type=_f32) + b_ref[...])


def _pre_linear(x, wT, b):
    return pl.pallas_call(
        _lin_body,
        grid=(N // BLK,),
        in_specs=[pl.BlockSpec((BLK, D), lambda i: (i, 0)),
                  pl.BlockSpec((D, D), lambda i: (0, 0)),
                  pl.BlockSpec((1, D), lambda i: (0, 0))],
        out_specs=pl.BlockSpec((BLK, D), lambda i: (i, 0)),
        out_shape=jax.ShapeDtypeStruct((N, D), _f32),
    )(x, wT, b)


def _sage_body(relu, norm, agg_ref, deg_ref, h_ref, wl_ref, bl_ref, wr_ref,
               o_ref):
    mean = agg_ref[...] / jnp.maximum(deg_ref[...], 1.0)
    o = (jnp.dot(mean, wl_ref[...], preferred_element_type=_f32)
         + bl_ref[...]
         + jnp.dot(h_ref[...], wr_ref[...], preferred_element_type=_f32))
    if relu:
        o = jnp.maximum(o, 0.0)
    if norm:
        nrm = jnp.sqrt(jnp.sum(o * o, axis=1, keepdims=True))
        o = o / jnp.maximum(nrm, 1e-12)
    o_ref[...] = o


def _sage(agg, deg, h, wlT, bl, wrT, relu, norm):
    return pl.pallas_call(
        functools.partial(_sage_body, relu, norm),
        grid=(N // BLK,),
        in_specs=[pl.BlockSpec((BLK, D), lambda i: (i, 0)),
                  pl.BlockSpec((BLK, 1), lambda i: (i, 0)),
                  pl.BlockSpec((BLK, D), lambda i: (i, 0)),
                  pl.BlockSpec((D, D), lambda i: (0, 0)),
                  pl.BlockSpec((1, D), lambda i: (0, 0)),
                  pl.BlockSpec((D, D), lambda i: (0, 0))],
        out_specs=pl.BlockSpec((BLK, D), lambda i: (i, 0)),
        out_shape=jax.ShapeDtypeStruct((N, D), _f32),
    )(agg, deg, h, wlT, bl, wrT)


def kernel(x, edge_index, W_pre, b_pre, Wl1, bl1, Wr1, Wl2, bl2, Wr2):
    src = edge_index[0]
    dst = edge_index[1]
    lists, cov = _sc_bin(src, dst)
    (deg16,) = _sc_deg(lists, cov)
    deg = deg16[:, :1]
    h0 = _pre_linear(x, W_pre.T, b_pre.reshape(1, D))
    (agg1,) = _sc_agg(h0, lists, cov)
    h1 = _sage(agg1, deg, h0, Wl1.T, bl1.reshape(1, D), Wr1.T,
               relu=True, norm=False)
    (agg2,) = _sc_agg(h1, lists, cov)
    out = _sage(agg2, deg, h1, Wl2.T, bl2.reshape(1, D), Wr2.T,
                relu=False, norm=True)
    return out


# speedup vs baseline: 1.1560x; 1.1560x over previous
import functools

import jax
import jax.numpy as jnp
from jax import lax
from jax.experimental import pallas as pl
from jax.experimental.pallas import tpu as pltpu
from jax.experimental.pallas import tpu_sc as plsc

N, E, D = 10000, 160000, 256
NC, NS, L = 2, 16, 16
NW = NC * NS
EPT = E // NW
NG = 313
RPT = 312
MAGIC = 3361
LAST_R = N - (NW - 1) * RPT
ACC_R = 330
TRASH_LOC = 328
NBK = NW + 1
CW = 48
BQ = 32
CAP_W = 6560
PKS = 9

_f32 = jnp.float32
_i32 = jnp.int32

_GDN = lax.GatherDimensionNumbers(offset_dims=(), collapsed_slice_dims=(0,),
                                  start_index_map=(0,))


def _vgather(vec, idx):
    return lax.gather(vec, idx[:, None], _GDN, (1,),
                      mode=lax.GatherScatterMode.PROMISE_IN_BOUNDS)


def _rquant(cnt):
    return ((cnt + BQ + 15) >> 5) << 5


def _sc_bin_body(src_hbm, dst_hbm, lx_hbm, cov_hbm,
                 sv, dv, covv, listv):
    c = lax.axis_index("c")
    s = lax.axis_index("s")
    w = c * NS + s
    ebase = w * EPT

    lane = lax.iota(_i32, L)
    zero16 = jnp.zeros((L,), _i32)

    pltpu.sync_copy(src_hbm.at[pl.ds(pl.multiple_of(ebase, 8), EPT)],
                    sv.at[pl.ds(0, EPT)])
    pltpu.sync_copy(dst_hbm.at[pl.ds(pl.multiple_of(ebase, 8), EPT)],
                    dv.at[pl.ds(0, EPT)])

    def bucket_of(g, d):
        b = jnp.minimum((d * MAGIC) >> 20, NW - 1)
        gflag = jnp.where(g == NG - 1, 1, 0)
        tail = jnp.where(lane >= 8, gflag, 0)
        return jnp.where(tail > 0, NW, b)

    def hist_add(b, h0, h1, h2):
        for e in range(L):
            bc = jnp.full((L,), b[e], _i32)
            h0 = h0 + jnp.where(lane == bc, 1, 0)
            h1 = h1 + jnp.where(lane + L == bc, 1, 0)
            h2 = h2 + jnp.where(lane + 2 * L == bc, 1, 0)
        return h0, h1, h2

    def pass_a(g, carry):
        h0, h1, h2 = carry
        d = dv[pl.ds(g * L, L)]
        b = bucket_of(g, d)
        return hist_add(b, h0, h1, h2)

    c0, c1, c2 = lax.fori_loop(0, NG, pass_a, (zero16, zero16, zero16))

    cvs = [c0, c1, c2]
    offs = []
    off_acc = jnp.int32(0)
    for b in range(NBK):
        offs.append(off_acc)
        off_acc = off_acc + _rquant(cvs[b // L][b % L])

    o0, o1, o2 = zero16, zero16, zero16
    for b in range(NBK):
        sel = jnp.where(lane == (b % L), offs[b], 0)
        if b // L == 0:
            o0 = o0 + sel
        elif b // L == 1:
            o1 = o1 + sel
        else:
            o2 = o2 + sel

    covv[pl.ds(0, L)] = (o0 << 16) | c0
    covv[pl.ds(L, L)] = (o1 << 16) | c1
    covv[pl.ds(2 * L, L)] = (o2 << 16) | c2
    pltpu.sync_copy(covv, cov_hbm.at[pl.ds(w * CW, CW)])

    def pass_b(g, carry):
        r0, r1, r2 = carry
        d = dv[pl.ds(g * L, L)]
        sc = sv[pl.ds(g * L, L)]
        b = bucket_of(g, d)
        bi = b & (L - 1)
        pk = (sc << PKS) | ((d - b * RPT) & ((1 << PKS) - 1))
        s0 = _vgather(r0, bi)
        s1 = _vgather(r1, bi)
        s2 = _vgather(r2, bi)
        slot = jnp.where(b < L, s0, jnp.where(b < 2 * L, s1, s2))
        t0 = _vgather(o0, bi)
        t1 = _vgather(o1, bi)
        t2 = _vgather(o2, bi)
        obase = jnp.where(b < L, t0, jnp.where(b < 2 * L, t1, t2))
        adj = zero16
        for e in range(L):
            bc = jnp.full((L,), b[e], _i32)
            adj = adj + jnp.where((b == bc) & (lane > e), 1, 0)
        pos = obase + slot + adj
        for e in range(L):
            listv[pl.ds(pos[e], L)] = jnp.full((L,), pk[e], _i32)
        r0, r1, r2 = hist_add(b, r0, r1, r2)
        return (r0, r1, r2)

    r0, r1, r2 = lax.fori_loop(0, NG, pass_b, (zero16, zero16, zero16))

    trash16 = jnp.full((L,), TRASH_LOC, _i32)
    rvs = [r0, r1, r2]
    for b in range(NBK):
        cnt_b = rvs[b // L][b % L]
        o_b = offs[b]
        oend = o_b + _rquant(cnt_b)
        pos0 = o_b + cnt_b
        for k in range(5):
            p2 = pos0 + k * L

            @pl.when(p2 <= oend - L)
            def _():
                listv[pl.ds(p2, L)] = trash16
        listv[pl.ds(oend - L, L)] = trash16

    pltpu.sync_copy(listv.at[pl.ds(0, CAP_W)],
                    lx_hbm.at[pl.ds(w * CAP_W, CAP_W)])


def _make_sc_bin():
    mesh = plsc.VectorSubcoreMesh(core_axis_name="c", subcore_axis_name="s")
    return pl.kernel(
        _sc_bin_body,
        out_type=(jax.ShapeDtypeStruct((NW * CAP_W,), _i32),
                  jax.ShapeDtypeStruct((NW * CW,), _i32)),
        mesh=mesh,
        scratch_types=(pltpu.VMEM((EPT + L,), _i32),
                       pltpu.VMEM((EPT + L,), _i32),
                       pltpu.VMEM((CW,), _i32),
                       pltpu.VMEM((CAP_W + L,), _i32)))


_sc_bin = _make_sc_bin()


def _sc_deg_body(lx_hbm, cov_hbm, deg_hbm, dacc, pv, locb, cov):
    c = lax.axis_index("c")
    s = lax.axis_index("s")
    w = c * NS + s
    zero16 = jnp.zeros((L,), _f32)
    one16 = jnp.ones((L,), _f32)

    def zd(i, _):
        dacc[i, :] = zero16
        return 0
    lax.fori_loop(0, ACC_R, zd, 0)

    pltpu.sync_copy(cov_hbm, cov.at[pl.ds(0, NW * CW)])

    for wsrc in range(NW):
        v = cov[pl.ds(wsrc * CW + w, L)][0]
        cnt = v & 0xFFFF
        off = v >> 16
        nb = jnp.where(cnt > 0, (cnt + BQ + 15) >> 5, 0)

        def batch(k, _):
            p0 = pl.multiple_of(wsrc * CAP_W + off + k * BQ, 8)
            pltpu.sync_copy(lx_hbm.at[pl.ds(p0, BQ)], pv)
            for g in range(BQ // L):
                vv = pv[pl.ds(g * L, L)]
                locb[pl.ds(g * L, L)] = vv & ((1 << PKS) - 1)

            def grp(g, _):
                lv = locb[pl.ds(g * L, L)]
                for l in range(L):
                    plsc.addupdate(dacc.at[lv[l]], one16)
                return 0
            lax.fori_loop(0, BQ // L, grp, 0)
            return 0
        lax.fori_loop(0, nb, batch, 0)

    obase = pl.multiple_of(w * RPT, 8)

    @pl.when(w < NW - 1)
    def _():
        pltpu.sync_copy(dacc.at[pl.ds(0, RPT)], deg_hbm.at[pl.ds(obase, RPT)])

    @pl.when(w == NW - 1)
    def _():
        pltpu.sync_copy(dacc.at[pl.ds(0, LAST_R)],
                        deg_hbm.at[pl.ds(obase, LAST_R)])


def _make_sc_deg():
    mesh = plsc.VectorSubcoreMesh(core_axis_name="c", subcore_axis_name="s")
    return pl.kernel(
        _sc_deg_body,
        out_type=(jax.ShapeDtypeStruct((N, L), _f32),),
        mesh=mesh,
        scratch_types=(pltpu.VMEM((ACC_R, L), _f32),
                       pltpu.VMEM((BQ,), _i32),
                       pltpu.VMEM((BQ,), _i32),
                       pltpu.VMEM((NW * CW + L,), _i32)))


_sc_deg = _make_sc_deg()


def _sc_agg_body(h_hbm, lx_hbm, cov_hbm, a0_hbm, a1_hbm,
                 acc0, acc1, rows, pv, srcb, locb, cov):
    c = lax.axis_index("c")
    s = lax.axis_index("s")
    w = c * NS + s
    accs = (acc0, acc1)
    outs = (a0_hbm, a1_hbm)

    zero16 = jnp.zeros((L,), _f32)

    for acc in accs:
        def za(i, _):
            acc[i >> 3, pl.ds((i & 7) * L, L)] = zero16
            return 0
        lax.fori_loop(0, ACC_R * 8, za, 0)

    pltpu.sync_copy(cov_hbm, cov.at[pl.ds(0, NW * CW)])

    def per_src(wsrc, _):
        v = cov[pl.ds(wsrc * CW + w, L)][0]
        cnt = v & 0xFFFF
        off = v >> 16
        nb = jnp.where(cnt > 0, (cnt + BQ + 15) >> 5, 0)

        def batch(k, _):
            p0 = pl.multiple_of(wsrc * CAP_W + off + k * BQ, 8)
            pltpu.sync_copy(lx_hbm.at[pl.ds(p0, BQ)], pv)
            for g in range(BQ // L):
                vv = pv[pl.ds(g * L, L)]
                srcb[pl.ds(g * L, L)] = vv >> PKS
                locb[pl.ds(g * L, L)] = vv & ((1 << PKS) - 1)
            pltpu.sync_copy(h_hbm.at[srcb], rows)

            def grp(g, _):
                lv = locb[pl.ds(g * L, L)]
                for l in range(L):
                    r = lv[l]
                    for p in range(8):
                        for q in range(2):
                            j = q * 8 + p
                            plsc.addupdate(accs[q].at[r, pl.ds(p * L, L)],
                                           rows[g * L + l, pl.ds(j * L, L)])
                return 0
            lax.fori_loop(0, BQ // L, grp, 0)
            return 0
        lax.fori_loop(0, nb, batch, 0)
        return 0
    lax.fori_loop(0, NW, per_src, 0)

    obase = pl.multiple_of(w * RPT, 8)

    @pl.when(w < NW - 1)
    def _():
        for acc, out in zip(accs, outs):
            pltpu.sync_copy(acc.at[pl.ds(0, RPT)], out.at[pl.ds(obase, RPT)])

    @pl.when(w == NW - 1)
    def _():
        for acc, out in zip(accs, outs):
            pltpu.sync_copy(acc.at[pl.ds(0, LAST_R)],
                            out.at[pl.ds(obase, LAST_R)])


def _make_sc_agg():
    mesh = plsc.VectorSubcoreMesh(core_axis_name="c", subcore_axis_name="s")
    DQ = D // 2
    return pl.kernel(
        _sc_agg_body,
        out_type=tuple(jax.ShapeDtypeStruct((N, DQ), _f32) for _ in range(2)),
        mesh=mesh,
        scratch_types=(pltpu.VMEM((ACC_R, DQ), _f32),
                       pltpu.VMEM((ACC_R, DQ), _f32),
                       pltpu.VMEM((BQ, D), _f32),
                       pltpu.VMEM((BQ,), _i32),
                       pltpu.VMEM((BQ,), _i32),
                       pltpu.VMEM((BQ,), _i32),
                       pltpu.VMEM((NW * CW + L,), _i32)))


_sc_agg = _make_sc_agg()

BLK = 1000


def _lin_body(x_ref, w_ref, b_ref, o_ref):
    o_ref[...] = (jnp.dot(x_ref[...], w_ref[...],
                          preferred_element_type=_f32) + b_ref[...])


def _pre_linear(x, wT, b):
    return pl.pallas_call(
        _lin_body,
        grid=(N // BLK,),
        in_specs=[pl.BlockSpec((BLK, D), lambda i: (i, 0)),
                  pl.BlockSpec((D, D), lambda i: (0, 0)),
                  pl.BlockSpec((1, D), lambda i: (0, 0))],
        out_specs=pl.BlockSpec((BLK, D), lambda i: (i, 0)),
        out_shape=jax.ShapeDtypeStruct((N, D), _f32),
    )(x, wT, b)


def _sage_body(relu, norm, a0_ref, a1_ref, deg_ref, h_ref,
               wl_ref, bl_ref, wr_ref, o_ref):
    agg = jnp.concatenate([a0_ref[...], a1_ref[...]], axis=1)
    mean = agg / jnp.maximum(deg_ref[...], 1.0)
    o = (jnp.dot(mean, wl_ref[...], preferred_element_type=_f32)
         + bl_ref[...]
         + jnp.dot(h_ref[...], wr_ref[...], preferred_element_type=_f32))
    if relu:
        o = jnp.maximum(o, 0.0)
    if norm:
        nrm = jnp.sqrt(jnp.sum(o * o, axis=1, keepdims=True))
        o = o / jnp.maximum(nrm, 1e-12)
    o_ref[...] = o


def _sage(aggq, deg, h, wlT, bl, wrT, relu, norm):
    DQ = D // 2
    return pl.pallas_call(
        functools.partial(_sage_body, relu, norm),
        grid=(N // BLK,),
        in_specs=[pl.BlockSpec((BLK, DQ), lambda i: (i, 0)),
                  pl.BlockSpec((BLK, DQ), lambda i: (i, 0)),
                  pl.BlockSpec((BLK, 1), lambda i: (i, 0)),
                  pl.BlockSpec((BLK, D), lambda i: (i, 0)),
                  pl.BlockSpec((D, D), lambda i: (0, 0)),
                  pl.BlockSpec((1, D), lambda i: (0, 0)),
                  pl.BlockSpec((D, D), lambda i: (0, 0))],
        out_specs=pl.BlockSpec((BLK, D), lambda i: (i, 0)),
        out_shape=jax.ShapeDtypeStruct((N, D), _f32),
    )(*aggq, deg, h, wlT, bl, wrT)


def kernel(x, edge_index, W_pre, b_pre, Wl1, bl1, Wr1, Wl2, bl2, Wr2):
    src = edge_index[0]
    dst = edge_index[1]
    lists, cov = _sc_bin(src, dst)
    (deg16,) = _sc_deg(lists, cov)
    deg = deg16[:, :1]
    h0 = _pre_linear(x, W_pre.T, b_pre.reshape(1, D))
    agg1 = _sc_agg(h0, lists, cov)
    h1 = _sage(agg1, deg, h0, Wl1.T, bl1.reshape(1, D), Wr1.T,
               relu=True, norm=False)
    agg2 = _sc_agg(h1, lists, cov)
    out = _sage(agg2, deg, h1, Wl2.T, bl2.reshape(1, D), Wr2.T,
                relu=False, norm=True)
    return out


# speedup vs baseline: 1.1620x; 1.0052x over previous
import functools

import jax
import jax.numpy as jnp
from jax import lax
from jax.experimental import pallas as pl
from jax.experimental.pallas import tpu as pltpu
from jax.experimental.pallas import tpu_sc as plsc

N, E, D = 10000, 160000, 256
NC, NS, L = 2, 16, 16
NW = NC * NS
EPT = E // NW
NG = 313
RPT = 312
MAGIC = 3361
LAST_R = N - (NW - 1) * RPT
ACC_R = 330
TRASH_LOC = 328
NBK = NW + 1
CW = 48
BQ = 32
CAP_W = 6560
PKS = 9

_f32 = jnp.float32
_i32 = jnp.int32

_GDN = lax.GatherDimensionNumbers(offset_dims=(), collapsed_slice_dims=(0,),
                                  start_index_map=(0,))


def _vgather(vec, idx):
    return lax.gather(vec, idx[:, None], _GDN, (1,),
                      mode=lax.GatherScatterMode.PROMISE_IN_BOUNDS)


def _rquant(cnt):
    return ((cnt + BQ + 15) >> 5) << 5


def _sc_bin_body(src_hbm, dst_hbm, lx_hbm, cov_hbm,
                 sv, dv, covv, listv):
    c = lax.axis_index("c")
    s = lax.axis_index("s")
    w = c * NS + s
    ebase = w * EPT

    lane = lax.iota(_i32, L)
    zero16 = jnp.zeros((L,), _i32)

    pltpu.sync_copy(src_hbm.at[pl.ds(pl.multiple_of(ebase, 8), EPT)],
                    sv.at[pl.ds(0, EPT)])
    pltpu.sync_copy(dst_hbm.at[pl.ds(pl.multiple_of(ebase, 8), EPT)],
                    dv.at[pl.ds(0, EPT)])

    def bucket_of(g, d):
        b = jnp.minimum((d * MAGIC) >> 20, NW - 1)
        gflag = jnp.where(g == NG - 1, 1, 0)
        tail = jnp.where(lane >= 8, gflag, 0)
        return jnp.where(tail > 0, NW, b)

    def hist_add(b, h0, h1, h2):
        for e in range(L):
            bc = jnp.full((L,), b[e], _i32)
            h0 = h0 + jnp.where(lane == bc, 1, 0)
            h1 = h1 + jnp.where(lane + L == bc, 1, 0)
            h2 = h2 + jnp.where(lane + 2 * L == bc, 1, 0)
        return h0, h1, h2

    def pass_a(g, carry):
        h0, h1, h2 = carry
        d = dv[pl.ds(g * L, L)]
        b = bucket_of(g, d)
        return hist_add(b, h0, h1, h2)

    c0, c1, c2 = lax.fori_loop(0, NG, pass_a, (zero16, zero16, zero16))

    cvs = [c0, c1, c2]
    offs = []
    off_acc = jnp.int32(0)
    for b in range(NBK):
        offs.append(off_acc)
        off_acc = off_acc + _rquant(cvs[b // L][b % L])

    o0, o1, o2 = zero16, zero16, zero16
    for b in range(NBK):
        sel = jnp.where(lane == (b % L), offs[b], 0)
        if b // L == 0:
            o0 = o0 + sel
        elif b // L == 1:
            o1 = o1 + sel
        else:
            o2 = o2 + sel

    covv[pl.ds(0, L)] = (o0 << 16) | c0
    covv[pl.ds(L, L)] = (o1 << 16) | c1
    covv[pl.ds(2 * L, L)] = (o2 << 16) | c2
    pltpu.sync_copy(covv, cov_hbm.at[pl.ds(w * CW, CW)])

    def pass_b(g, carry):
        r0, r1, r2 = carry
        d = dv[pl.ds(g * L, L)]
        sc = sv[pl.ds(g * L, L)]
        b = bucket_of(g, d)
        bi = b & (L - 1)
        pk = (sc << PKS) | ((d - b * RPT) & ((1 << PKS) - 1))
        s0 = _vgather(r0, bi)
        s1 = _vgather(r1, bi)
        s2 = _vgather(r2, bi)
        slot = jnp.where(b < L, s0, jnp.where(b < 2 * L, s1, s2))
        t0 = _vgather(o0, bi)
        t1 = _vgather(o1, bi)
        t2 = _vgather(o2, bi)
        obase = jnp.where(b < L, t0, jnp.where(b < 2 * L, t1, t2))
        adj = zero16
        for e in range(L):
            bc = jnp.full((L,), b[e], _i32)
            adj = adj + jnp.where((b == bc) & (lane > e), 1, 0)
        pos = obase + slot + adj
        for e in range(L):
            listv[pl.ds(pos[e], L)] = jnp.full((L,), pk[e], _i32)
        r0, r1, r2 = hist_add(b, r0, r1, r2)
        return (r0, r1, r2)

    r0, r1, r2 = lax.fori_loop(0, NG, pass_b, (zero16, zero16, zero16))

    trash16 = jnp.full((L,), TRASH_LOC, _i32)
    rvs = [r0, r1, r2]
    for b in range(NBK):
        cnt_b = rvs[b // L][b % L]
        o_b = offs[b]
        oend = o_b + _rquant(cnt_b)
        pos0 = o_b + cnt_b
        for k in range(5):
            p2 = pos0 + k * L

            @pl.when(p2 <= oend - L)
            def _():
                listv[pl.ds(p2, L)] = trash16
        listv[pl.ds(oend - L, L)] = trash16

    pltpu.sync_copy(listv.at[pl.ds(0, CAP_W)],
                    lx_hbm.at[pl.ds(w * CAP_W, CAP_W)])


def _make_sc_bin():
    mesh = plsc.VectorSubcoreMesh(core_axis_name="c", subcore_axis_name="s")
    return pl.kernel(
        _sc_bin_body,
        out_type=(jax.ShapeDtypeStruct((NW * CAP_W,), _i32),
                  jax.ShapeDtypeStruct((NW * CW,), _i32)),
        mesh=mesh,
        scratch_types=(pltpu.VMEM((EPT + L,), _i32),
                       pltpu.VMEM((EPT + L,), _i32),
                       pltpu.VMEM((CW,), _i32),
                       pltpu.VMEM((CAP_W + L,), _i32)))


_sc_bin = _make_sc_bin()


def _sc_deg_body(lx_hbm, cov_hbm, deg_hbm, dacc, pv, locb, cov):
    c = lax.axis_index("c")
    s = lax.axis_index("s")
    w = c * NS + s
    zero16 = jnp.zeros((L,), _f32)
    one16 = jnp.ones((L,), _f32)

    def zd(i, _):
        dacc[i, :] = zero16
        return 0
    lax.fori_loop(0, ACC_R, zd, 0)

    pltpu.sync_copy(cov_hbm, cov.at[pl.ds(0, NW * CW)])

    for wsrc in range(NW):
        v = cov[pl.ds(wsrc * CW + w, L)][0]
        cnt = v & 0xFFFF
        off = v >> 16
        nb = jnp.where(cnt > 0, (cnt + BQ + 15) >> 5, 0)

        def batch(k, _):
            p0 = pl.multiple_of(wsrc * CAP_W + off + k * BQ, 8)
            pltpu.sync_copy(lx_hbm.at[pl.ds(p0, BQ)], pv)
            for g in range(BQ // L):
                vv = pv[pl.ds(g * L, L)]
                locb[pl.ds(g * L, L)] = vv & ((1 << PKS) - 1)

            def grp(g, _):
                lv = locb[pl.ds(g * L, L)]
                for l in range(L):
                    plsc.addupdate(dacc.at[lv[l]], one16)
                return 0
            lax.fori_loop(0, BQ // L, grp, 0)
            return 0
        lax.fori_loop(0, nb, batch, 0)

    obase = pl.multiple_of(w * RPT, 8)

    @pl.when(w < NW - 1)
    def _():
        pltpu.sync_copy(dacc.at[pl.ds(0, RPT)], deg_hbm.at[pl.ds(obase, RPT)])

    @pl.when(w == NW - 1)
    def _():
        pltpu.sync_copy(dacc.at[pl.ds(0, LAST_R)],
                        deg_hbm.at[pl.ds(obase, LAST_R)])


def _make_sc_deg():
    mesh = plsc.VectorSubcoreMesh(core_axis_name="c", subcore_axis_name="s")
    return pl.kernel(
        _sc_deg_body,
        out_type=(jax.ShapeDtypeStruct((N, L), _f32),),
        mesh=mesh,
        scratch_types=(pltpu.VMEM((ACC_R, L), _f32),
                       pltpu.VMEM((BQ,), _i32),
                       pltpu.VMEM((BQ,), _i32),
                       pltpu.VMEM((NW * CW + L,), _i32)))


_sc_deg = _make_sc_deg()


def _sc_agg_body(h_hbm, lx_hbm, cov_hbm, a0_hbm, a1_hbm,
                 acc0, acc1, rows0, rows1, pv, srcb0, srcb1, locb0, locb1,
                 cov, sem0, sem1):
    c = lax.axis_index("c")
    s = lax.axis_index("s")
    w = c * NS + s
    accs = (acc0, acc1)
    outs = (a0_hbm, a1_hbm)
    rowsb = (rows0, rows1)
    srcbb = (srcb0, srcb1)
    locbb = (locb0, locb1)
    semb = (sem0, sem1)

    zero16 = jnp.zeros((L,), _f32)

    for acc in accs:
        def za(i, _):
            acc[i >> 3, pl.ds((i & 7) * L, L)] = zero16
            return 0
        lax.fori_loop(0, ACC_R * 8, za, 0)

    pltpu.sync_copy(cov_hbm, cov.at[pl.ds(0, NW * CW)])

    def load_unpack_start(wsrc, off, k, buf):
        p0 = pl.multiple_of(wsrc * CAP_W + off + k * BQ, 8)
        pltpu.sync_copy(lx_hbm.at[pl.ds(p0, BQ)], pv)
        for g in range(BQ // L):
            vv = pv[pl.ds(g * L, L)]
            srcbb[buf][pl.ds(g * L, L)] = vv >> PKS
            locbb[buf][pl.ds(g * L, L)] = vv & ((1 << PKS) - 1)
        pltpu.async_copy(h_hbm.at[srcbb[buf]], rowsb[buf], semb[buf])

    def accumulate(buf):
        def grp(g, _):
            lv = locbb[buf][pl.ds(g * L, L)]
            for l in range(L):
                r = lv[l]
                for p in range(8):
                    for q in range(2):
                        j = q * 8 + p
                        plsc.addupdate(accs[q].at[r, pl.ds(p * L, L)],
                                       rowsb[buf][g * L + l, pl.ds(j * L, L)])
            return 0
        lax.fori_loop(0, BQ // L, grp, 0)

    def per_src(wsrc, _):
        v = cov[pl.ds(wsrc * CW + w, L)][0]
        cnt = v & 0xFFFF
        off = v >> 16
        nb = jnp.where(cnt > 0, (cnt + BQ + 15) >> 5, 0)

        @pl.when(nb > 0)
        def _():
            load_unpack_start(wsrc, off, 0, 0)

            def pair(kk, _):
                k0 = kk * 2
                k1 = k0 + 1

                @pl.when(k1 < nb)
                def _():
                    load_unpack_start(wsrc, off, k1, 1)
                pltpu.make_async_copy(h_hbm.at[srcb0], rows0, sem0).wait()
                accumulate(0)

                @pl.when(k0 + 2 < nb)
                def _():
                    load_unpack_start(wsrc, off, k0 + 2, 0)

                @pl.when(k1 < nb)
                def _():
                    pltpu.make_async_copy(h_hbm.at[srcb1], rows1,
                                          sem1).wait()
                    accumulate(1)
                return 0
            lax.fori_loop(0, (nb + 1) >> 1, pair, 0)
        return 0
    lax.fori_loop(0, NW, per_src, 0)

    obase = pl.multiple_of(w * RPT, 8)

    @pl.when(w < NW - 1)
    def _():
        for acc, out in zip(accs, outs):
            pltpu.sync_copy(acc.at[pl.ds(0, RPT)], out.at[pl.ds(obase, RPT)])

    @pl.when(w == NW - 1)
    def _():
        for acc, out in zip(accs, outs):
            pltpu.sync_copy(acc.at[pl.ds(0, LAST_R)],
                            out.at[pl.ds(obase, LAST_R)])


def _make_sc_agg():
    mesh = plsc.VectorSubcoreMesh(core_axis_name="c", subcore_axis_name="s")
    DQ = D // 2
    return pl.kernel(
        _sc_agg_body,
        out_type=tuple(jax.ShapeDtypeStruct((N, DQ), _f32) for _ in range(2)),
        mesh=mesh,
        scratch_types=(pltpu.VMEM((ACC_R, DQ), _f32),
                       pltpu.VMEM((ACC_R, DQ), _f32),
                       pltpu.VMEM((BQ, D), _f32),
                       pltpu.VMEM((BQ, D), _f32),
                       pltpu.VMEM((BQ,), _i32),
                       pltpu.VMEM((BQ,), _i32),
                       pltpu.VMEM((BQ,), _i32),
                       pltpu.VMEM((BQ,), _i32),
                       pltpu.VMEM((BQ,), _i32),
                       pltpu.VMEM((NW * CW + L,), _i32),
                       pltpu.SemaphoreType.DMA,
                       pltpu.SemaphoreType.DMA))


_sc_agg = _make_sc_agg()

BLK = 1000


def _lin_body(x_ref, w_ref, b_ref, o_ref):
    o_ref[...] = (jnp.dot(x_ref[...], w_ref[...],
                          preferred_element_type=_f32) + b_ref[...])


def _pre_linear(x, wT, b):
    return pl.pallas_call(
        _lin_body,
        grid=(N // BLK,),
        in_specs=[pl.BlockSpec((BLK, D), lambda i: (i, 0)),
                  pl.BlockSpec((D, D), lambda i: (0, 0)),
                  pl.BlockSpec((1, D), lambda i: (0, 0))],
        out_specs=pl.BlockSpec((BLK, D), lambda i: (i, 0)),
        out_shape=jax.ShapeDtypeStruct((N, D), _f32),
    )(x, wT, b)


def _sage_body(relu, norm, a0_ref, a1_ref, deg_ref, h_ref,
               wl_ref, bl_ref, wr_ref, o_ref):
    agg = jnp.concatenate([a0_ref[...], a1_ref[...]], axis=1)
    mean = agg / jnp.maximum(deg_ref[...], 1.0)
    o = (jnp.dot(mean, wl_ref[...], preferred_element_type=_f32)
         + bl_ref[...]
         + jnp.dot(h_ref[...], wr_ref[...], preferred_element_type=_f32))
    if relu:
        o = jnp.maximum(o, 0.0)
    if norm:
        nrm = jnp.sqrt(jnp.sum(o * o, axis=1, keepdims=True))
        o = o / jnp.maximum(nrm, 1e-12)
    o_ref[...] = o


def _sage(aggq, deg, h, wlT, bl, wrT, relu, norm):
    DQ = D // 2
    return pl.pallas_call(
        functools.partial(_sage_body, relu, norm),
        grid=(N // BLK,),
        in_specs=[pl.BlockSpec((BLK, DQ), lambda i: (i, 0)),
                  pl.BlockSpec((BLK, DQ), lambda i: (i, 0)),
                  pl.BlockSpec((BLK, 1), lambda i: (i, 0)),
                  pl.BlockSpec((BLK, D), lambda i: (i, 0)),
                  pl.BlockSpec((D, D), lambda i: (0, 0)),
                  pl.BlockSpec((1, D), lambda i: (0, 0)),
                  pl.BlockSpec((D, D), lambda i: (0, 0))],
        out_specs=pl.BlockSpec((BLK, D), lambda i: (i, 0)),
        out_shape=jax.ShapeDtypeStruct((N, D), _f32),
    )(*aggq, deg, h, wlT, bl, wrT)


def kernel(x, edge_index, W_pre, b_pre, Wl1, bl1, Wr1, Wl2, bl2, Wr2):
    src = edge_index[0]
    dst = edge_index[1]
    lists, cov = _sc_bin(src, dst)
    (deg16,) = _sc_deg(lists, cov)
    deg = deg16[:, :1]
    h0 = _pre_linear(x, W_pre.T, b_pre.reshape(1, D))
    agg1 = _sc_agg(h0, lists, cov)
    h1 = _sage(agg1, deg, h0, Wl1.T, bl1.reshape(1, D), Wr1.T,
               relu=True, norm=False)
    agg2 = _sc_agg(h1, lists, cov)
    out = _sage(agg2, deg, h1, Wl2.T, bl2.reshape(1, D), Wr2.T,
                relu=False, norm=True)
    return out
